# baseline (speedup 1.0000x reference)
import functools

import jax
import jax.numpy as jnp
from jax import lax
from jax.experimental import pallas as pl
from jax.experimental.pallas import tpu as pltpu
from jax.experimental.pallas import tpu_sc as plsc

M = 1024
D = 256
N = 16384

NC = 2
NS = 16
L = 16
NW = NC * NS
RPW = N // NW
CH = 128
NCH = RPW // CH

TN = 4096
NT = N // TN


def _stats_body(keys_ref, q_ref, rowmax_ref, gidx_ref, colmax_ref):
    i = pl.program_id(0)
    s = lax.dot_general(
        keys_ref[...], q_ref[...],
        dimension_numbers=(((1,), (1,)), ((), ())),
        preferred_element_type=jnp.float32,
    )
    rm = jnp.max(s, axis=0)
    rows = lax.broadcasted_iota(jnp.int32, (M, TN), 0)
    g = jnp.min(jnp.where(s == rm[None, :], rows, M), axis=0)
    rowmax_ref[...] = rm.reshape(1, 1, TN)
    gidx_ref[...] = g.astype(jnp.int32).reshape(1, 1, TN)
    cm = jnp.max(s, axis=1).reshape(M, 1)

    @pl.when(i == 0)
    def _():
        colmax_ref[...] = cm

    @pl.when(i > 0)
    def _():
        colmax_ref[...] = jnp.maximum(colmax_ref[...], cm)


_stats_call = pl.pallas_call(
    _stats_body,
    grid=(NT,),
    in_specs=[
        pl.BlockSpec((M, D), lambda i: (0, 0)),
        pl.BlockSpec((TN, D), lambda i: (i, 0)),
    ],
    out_specs=[
        pl.BlockSpec((1, 1, TN), lambda i: (i, 0, 0)),
        pl.BlockSpec((1, 1, TN), lambda i: (i, 0, 0)),
        pl.BlockSpec((M, 1), lambda i: (0, 0)),
    ],
    out_shape=[
        jax.ShapeDtypeStruct((NT, 1, TN), jnp.float32),
        jax.ShapeDtypeStruct((NT, 1, TN), jnp.int32),
        jax.ShapeDtypeStruct((M, 1), jnp.float32),
    ],
)


def _weights_body(rowmax_ref, gidx_ref, colmax_ref, wrep_ref):
    rm = rowmax_ref[0, 0, :]
    g = gidx_ref[0, 0, :]
    rows = lax.broadcasted_iota(jnp.int32, (M, TN), 0)
    cm_bc = jnp.broadcast_to(colmax_ref[...], (M, TN))
    cmg = jnp.max(jnp.where(rows == g[None, :], cm_bc, -jnp.inf), axis=0)
    w = jnp.exp(rm - cmg)
    wrep_ref[...] = jnp.broadcast_to(w[:, None], (TN, L)).reshape(1, TN, L)


_weights_call = pl.pallas_call(
    _weights_body,
    grid=(NT,),
    in_specs=[
        pl.BlockSpec((1, 1, TN), lambda i: (i, 0, 0)),
        pl.BlockSpec((1, 1, TN), lambda i: (i, 0, 0)),
        pl.BlockSpec((M, 1), lambda i: (0, 0)),
    ],
    out_specs=pl.BlockSpec((1, TN, L), lambda i: (i, 0, 0)),
    out_shape=jax.ShapeDtypeStruct((NT, TN, L), jnp.float32),
)


DH = D // 2
ZR = 16


def _scatter_body(gidx_hbm, wrep_hbm, value_hbm, out_hbm,
                  wrep_a, wrep_b, v0a, v0b, v1a, v1b, idx_a, idx_b, zero_v,
                  acc0, acc1, sem_la, sem_lb, sem_sa, sem_sb):
    c = lax.axis_index("c")
    s = lax.axis_index("s")
    wid = s * NC + c
    base = wid * RPW
    arows = M // NS
    rows0 = s * arows

    def zbody(i, _):
        r = i // (D // L)
        k = i % (D // L)
        zero_v[r, pl.ds(k * L, L)] = jnp.zeros((L,), jnp.float32)
        return 0
    lax.fori_loop(0, ZR * (D // L), zbody, 0)
    for kk in range(arows // ZR):
        pltpu.sync_copy(zero_v.at[:, pl.ds(0, DH)],
                        acc0.at[pl.ds(rows0 + kk * ZR, ZR)])
        pltpu.sync_copy(zero_v.at[:, pl.ds(DH, DH)],
                        acc1.at[pl.ds(rows0 + kk * ZR, ZR)])

    banks = [(wrep_a, v0a, v1a, idx_a, sem_la, sem_sa),
             (wrep_b, v0b, v1b, idx_b, sem_lb, sem_sb)]

    def issue_loads(t, bank):
        wrep_v, v0, v1, idx_v, sem_l, _ = bank
        rbase = base + t * CH
        return [
            pltpu.async_copy(
                value_hbm.at[pl.ds(rbase, CH), pl.ds(0, DH)], v0, sem_l),
            pltpu.async_copy(
                value_hbm.at[pl.ds(rbase, CH), pl.ds(DH, DH)], v1, sem_l),
            pltpu.async_copy(wrep_hbm.at[pl.ds(rbase, CH)], wrep_v, sem_l),
            pltpu.async_copy(gidx_hbm.at[pl.ds(rbase, CH)], idx_v, sem_l),
        ]

    plsc.subcore_barrier()

    pend_loads = {0: issue_loads(0, banks[0])}
    pend_scat = {}
    for t in range(NCH):
        bank = banks[t % 2]
        wrep_v, v0, v1, idx_v, _, sem_s = bank
        for cpy in pend_loads.pop(t):
            cpy.wait()
        if t + 1 < NCH:
            nxt = banks[(t + 1) % 2]
            for cpy in pend_scat.pop(t - 1, ()):
                cpy.wait()
            pend_loads[t + 1] = issue_loads(t + 1, nxt)

        def rbody(r, _):
            wrow = wrep_v[r, :]
            for k in range(DH // L):
                v0[r, pl.ds(k * L, L)] = v0[r, pl.ds(k * L, L)] * wrow
                v1[r, pl.ds(k * L, L)] = v1[r, pl.ds(k * L, L)] * wrow
            return 0
        lax.fori_loop(0, CH, rbody, 0)

        pend_scat[t] = [
            pltpu.async_copy(v0, acc0.at[idx_v], sem_s, add=True),
            pltpu.async_copy(v1, acc1.at[idx_v], sem_s, add=True),
        ]

    for t in sorted(pend_scat):
        for cpy in pend_scat[t]:
            cpy.wait()

    plsc.subcore_barrier()
    pltpu.sync_copy(acc0.at[pl.ds(rows0, arows)],
                    out_hbm.at[c, pl.ds(rows0, arows), pl.ds(0, DH)])
    pltpu.sync_copy(acc1.at[pl.ds(rows0, arows)],
                    out_hbm.at[c, pl.ds(rows0, arows), pl.ds(DH, DH)])


_scatter_call = functools.partial(
    pl.kernel,
    out_type=jax.ShapeDtypeStruct((NC, M, D), jnp.float32),
    mesh=plsc.VectorSubcoreMesh(
        core_axis_name="c", subcore_axis_name="s",
        num_cores=NC, num_subcores=NS),
    scratch_types=[
        pltpu.VMEM((CH, L), jnp.float32),
        pltpu.VMEM((CH, L), jnp.float32),
        pltpu.VMEM((CH, DH), jnp.float32),
        pltpu.VMEM((CH, DH), jnp.float32),
        pltpu.VMEM((CH, DH), jnp.float32),
        pltpu.VMEM((CH, DH), jnp.float32),
        pltpu.VMEM((CH,), jnp.int32),
        pltpu.VMEM((CH,), jnp.int32),
        pltpu.VMEM((ZR, D), jnp.float32),
        pltpu.VMEM_SHARED((M, DH), jnp.float32),
        pltpu.VMEM_SHARED((M, DH), jnp.float32),
        pltpu.SemaphoreType.DMA,
        pltpu.SemaphoreType.DMA,
        pltpu.SemaphoreType.DMA,
        pltpu.SemaphoreType.DMA,
    ],
)(_scatter_body)


def _finalize_body(p_ref, keys_ref, out_ref):
    comb = p_ref[0] + p_ref[1] + keys_ref[...]
    nrm = jnp.sqrt(jnp.sum(comb * comb, axis=1, keepdims=True))
    out_ref[...] = comb / jnp.maximum(nrm, 1e-12)


_finalize_call = pl.pallas_call(
    _finalize_body,
    out_shape=jax.ShapeDtypeStruct((M, D), jnp.float32),
)


def kernel(keys, query, value):
    rowmax3, gidx3, colmax2 = _stats_call(keys, query)
    wrep3 = _weights_call(rowmax3, gidx3, colmax2)
    gidx = gidx3.reshape(N)
    wrep = wrep3.reshape(N, L)
    partials = _scatter_call(gidx, wrep, value)
    return _finalize_call(partials, keys)


# speedup vs baseline: 1.0054x; 1.0054x over previous
import functools

import jax
import jax.numpy as jnp
from jax import lax
from jax.experimental import pallas as pl
from jax.experimental.pallas import tpu as pltpu
from jax.experimental.pallas import tpu_sc as plsc

M = 1024
D = 256
N = 16384

NC = 2
NS = 16
L = 16
NW = NC * NS
RPW = N // NW
CH = 128
NCH = RPW // CH

TN = 2048
NT = N // TN


def _stats_body(keys_ref, q_ref, rowmax_ref, gidx_ref, colmax_ref):
    i = pl.program_id(0)
    s = lax.dot_general(
        keys_ref[...], q_ref[...],
        dimension_numbers=(((1,), (1,)), ((), ())),
        preferred_element_type=jnp.float32,
    )
    rm = jnp.max(s, axis=0)
    rows = lax.broadcasted_iota(jnp.int32, (M, TN), 0)
    g = jnp.min(jnp.where(s == rm[None, :], rows, M), axis=0)
    rowmax_ref[...] = rm.reshape(1, 1, TN)
    gidx_ref[...] = g.astype(jnp.int32).reshape(1, 1, TN)
    cm = jnp.max(s, axis=1).reshape(M, 1)

    @pl.when(i == 0)
    def _():
        colmax_ref[...] = cm

    @pl.when(i > 0)
    def _():
        colmax_ref[...] = jnp.maximum(colmax_ref[...], cm)


_stats_call = pl.pallas_call(
    _stats_body,
    grid=(NT,),
    in_specs=[
        pl.BlockSpec((M, D), lambda i: (0, 0)),
        pl.BlockSpec((TN, D), lambda i: (i, 0)),
    ],
    out_specs=[
        pl.BlockSpec((1, 1, TN), lambda i: (i, 0, 0)),
        pl.BlockSpec((1, 1, TN), lambda i: (i, 0, 0)),
        pl.BlockSpec((M, 1), lambda i: (0, 0)),
    ],
    out_shape=[
        jax.ShapeDtypeStruct((NT, 1, TN), jnp.float32),
        jax.ShapeDtypeStruct((NT, 1, TN), jnp.int32),
        jax.ShapeDtypeStruct((M, 1), jnp.float32),
    ],
)


def _weights_body(rowmax_ref, gidx_ref, colmax_ref, wrep_ref):
    rm = rowmax_ref[0, 0, :]
    g = gidx_ref[0, 0, :]
    rows = lax.broadcasted_iota(jnp.int32, (M, TN), 0)
    cm_bc = jnp.broadcast_to(colmax_ref[...], (M, TN))
    cmg = jnp.max(jnp.where(rows == g[None, :], cm_bc, -jnp.inf), axis=0)
    w = jnp.exp(rm - cmg)
    wrep_ref[...] = jnp.broadcast_to(w[:, None], (TN, L)).reshape(1, TN, L)


_weights_call = pl.pallas_call(
    _weights_body,
    grid=(NT,),
    in_specs=[
        pl.BlockSpec((1, 1, TN), lambda i: (i, 0, 0)),
        pl.BlockSpec((1, 1, TN), lambda i: (i, 0, 0)),
        pl.BlockSpec((M, 1), lambda i: (0, 0)),
    ],
    out_specs=pl.BlockSpec((1, TN, L), lambda i: (i, 0, 0)),
    out_shape=jax.ShapeDtypeStruct((NT, TN, L), jnp.float32),
)


DH = D // 2
ZR = 16


def _scatter_body(gidx_hbm, wrep_hbm, value_hbm, out_hbm,
                  wrep_a, wrep_b, v0a, v0b, v1a, v1b, idx_a, idx_b, zero_v,
                  acc0, acc1, sem_la, sem_lb, sem_sa, sem_sb):
    c = lax.axis_index("c")
    s = lax.axis_index("s")
    wid = s * NC + c
    base = wid * RPW
    arows = M // NS
    rows0 = s * arows

    def zbody(i, _):
        r = i // (D // L)
        k = i % (D // L)
        zero_v[r, pl.ds(k * L, L)] = jnp.zeros((L,), jnp.float32)
        return 0
    lax.fori_loop(0, ZR * (D // L), zbody, 0)
    for kk in range(arows // ZR):
        pltpu.sync_copy(zero_v.at[:, pl.ds(0, DH)],
                        acc0.at[pl.ds(rows0 + kk * ZR, ZR)])
        pltpu.sync_copy(zero_v.at[:, pl.ds(DH, DH)],
                        acc1.at[pl.ds(rows0 + kk * ZR, ZR)])

    banks = [(wrep_a, v0a, v1a, idx_a, sem_la, sem_sa),
             (wrep_b, v0b, v1b, idx_b, sem_lb, sem_sb)]

    def issue_loads(t, bank):
        wrep_v, v0, v1, idx_v, sem_l, _ = bank
        rbase = base + t * CH
        return [
            pltpu.async_copy(
                value_hbm.at[pl.ds(rbase, CH), pl.ds(0, DH)], v0, sem_l),
            pltpu.async_copy(
                value_hbm.at[pl.ds(rbase, CH), pl.ds(DH, DH)], v1, sem_l),
            pltpu.async_copy(wrep_hbm.at[pl.ds(rbase, CH)], wrep_v, sem_l),
            pltpu.async_copy(gidx_hbm.at[pl.ds(rbase, CH)], idx_v, sem_l),
        ]

    plsc.subcore_barrier()

    pend_loads = {0: issue_loads(0, banks[0])}
    pend_scat = {}
    for t in range(NCH):
        bank = banks[t % 2]
        wrep_v, v0, v1, idx_v, _, sem_s = bank
        for cpy in pend_loads.pop(t):
            cpy.wait()
        if t + 1 < NCH:
            nxt = banks[(t + 1) % 2]
            for cpy in pend_scat.pop(t - 1, ()):
                cpy.wait()
            pend_loads[t + 1] = issue_loads(t + 1, nxt)

        def rbody(r, _):
            wrow = wrep_v[r, :]
            for k in range(DH // L):
                v0[r, pl.ds(k * L, L)] = v0[r, pl.ds(k * L, L)] * wrow
                v1[r, pl.ds(k * L, L)] = v1[r, pl.ds(k * L, L)] * wrow
            return 0
        lax.fori_loop(0, CH, rbody, 0)

        pend_scat[t] = [
            pltpu.async_copy(v0, acc0.at[idx_v], sem_s, add=True),
            pltpu.async_copy(v1, acc1.at[idx_v], sem_s, add=True),
        ]

    for t in sorted(pend_scat):
        for cpy in pend_scat[t]:
            cpy.wait()

    plsc.subcore_barrier()
    pltpu.sync_copy(acc0.at[pl.ds(rows0, arows)],
                    out_hbm.at[c, pl.ds(rows0, arows), pl.ds(0, DH)])
    pltpu.sync_copy(acc1.at[pl.ds(rows0, arows)],
                    out_hbm.at[c, pl.ds(rows0, arows), pl.ds(DH, DH)])


_scatter_call = functools.partial(
    pl.kernel,
    out_type=jax.ShapeDtypeStruct((NC, M, D), jnp.float32),
    mesh=plsc.VectorSubcoreMesh(
        core_axis_name="c", subcore_axis_name="s",
        num_cores=NC, num_subcores=NS),
    scratch_types=[
        pltpu.VMEM((CH, L), jnp.float32),
        pltpu.VMEM((CH, L), jnp.float32),
        pltpu.VMEM((CH, DH), jnp.float32),
        pltpu.VMEM((CH, DH), jnp.float32),
        pltpu.VMEM((CH, DH), jnp.float32),
        pltpu.VMEM((CH, DH), jnp.float32),
        pltpu.VMEM((CH,), jnp.int32),
        pltpu.VMEM((CH,), jnp.int32),
        pltpu.VMEM((ZR, D), jnp.float32),
        pltpu.VMEM_SHARED((M, DH), jnp.float32),
        pltpu.VMEM_SHARED((M, DH), jnp.float32),
        pltpu.SemaphoreType.DMA,
        pltpu.SemaphoreType.DMA,
        pltpu.SemaphoreType.DMA,
        pltpu.SemaphoreType.DMA,
    ],
)(_scatter_body)


def _finalize_body(p_ref, keys_ref, out_ref):
    comb = p_ref[0] + p_ref[1] + keys_ref[...]
    nrm = jnp.sqrt(jnp.sum(comb * comb, axis=1, keepdims=True))
    out_ref[...] = comb / jnp.maximum(nrm, 1e-12)


_finalize_call = pl.pallas_call(
    _finalize_body,
    out_shape=jax.ShapeDtypeStruct((M, D), jnp.float32),
)


def kernel(keys, query, value):
    rowmax3, gidx3, colmax2 = _stats_call(keys, query)
    wrep3 = _weights_call(rowmax3, gidx3, colmax2)
    gidx = gidx3.reshape(N)
    wrep = wrep3.reshape(N, L)
    partials = _scatter_call(gidx, wrep, value)
    return _finalize_call(partials, keys)


# speedup vs baseline: 1.0229x; 1.0173x over previous
import functools

import jax
import jax.numpy as jnp
from jax import lax
from jax.experimental import pallas as pl
from jax.experimental.pallas import tpu as pltpu
from jax.experimental.pallas import tpu_sc as plsc

M = 1024
D = 256
N = 16384

NC = 2
NS = 16
L = 16
NW = NC * NS
RPW = N // NW
CH = 128
NCH = RPW // CH

TN = 2048
NT = N // TN


def _stats_body(keys_ref, q_ref, gidx_ref, wrep_ref, rm_s, g_s, cm_s):
    i = pl.program_id(0)

    @pl.when(i < NT)
    def _():
        sc = lax.dot_general(
            keys_ref[...], q_ref[...],
            dimension_numbers=(((1,), (1,)), ((), ())),
            preferred_element_type=jnp.float32,
        )
        rm = jnp.max(sc, axis=0)
        rows = lax.broadcasted_iota(jnp.int32, (M, TN), 0)
        g = jnp.min(jnp.where(sc == rm[None, :], rows, M), axis=0)
        gi = g.astype(jnp.int32)
        gidx_ref[...] = gi.reshape(1, 1, TN)
        rm_s[i, :] = rm
        g_s[i, :] = gi
        cm = jnp.max(sc, axis=1).reshape(M, 1)

        @pl.when(i == 0)
        def _():
            cm_s[...] = cm

        @pl.when(i > 0)
        def _():
            cm_s[...] = jnp.maximum(cm_s[...], cm)

    @pl.when(i >= NT)
    def _():
        t = i - NT
        rm = rm_s[t, :]
        g = g_s[t, :]
        rows = lax.broadcasted_iota(jnp.int32, (M, TN), 0)
        cm_bc = jnp.broadcast_to(cm_s[...], (M, TN))
        cmg = jnp.max(jnp.where(rows == g[None, :], cm_bc, -jnp.inf), axis=0)
        w = jnp.exp(rm - cmg)
        wrep_ref[...] = jnp.broadcast_to(w[:, None], (TN, L)).reshape(1, TN, L)


_stats_call = pl.pallas_call(
    _stats_body,
    grid=(2 * NT,),
    in_specs=[
        pl.BlockSpec((M, D), lambda i: (0, 0)),
        pl.BlockSpec((TN, D), lambda i: (jnp.minimum(i, NT - 1), 0)),
    ],
    out_specs=[
        pl.BlockSpec((1, 1, TN),
                     lambda i: (jnp.minimum(i, NT - 1), 0, 0)),
        pl.BlockSpec((1, TN, L),
                     lambda i: (jnp.maximum(i - NT, 0), 0, 0)),
    ],
    out_shape=[
        jax.ShapeDtypeStruct((NT, 1, TN), jnp.int32),
        jax.ShapeDtypeStruct((NT, TN, L), jnp.float32),
    ],
    scratch_shapes=[
        pltpu.VMEM((NT, TN), jnp.float32),
        pltpu.VMEM((NT, TN), jnp.int32),
        pltpu.VMEM((M, 1), jnp.float32),
    ],
)


DH = D // 2
ZR = 16


def _scatter_body(gidx_hbm, wrep_hbm, value_hbm, out_hbm,
                  wrep_a, wrep_b, v0a, v0b, v1a, v1b, idx_a, idx_b, zero_v,
                  acc0, acc1, sem_la, sem_lb, sem_sa, sem_sb):
    c = lax.axis_index("c")
    s = lax.axis_index("s")
    wid = s * NC + c
    base = wid * RPW
    arows = M // NS
    rows0 = s * arows

    def zbody(i, _):
        r = i // (D // L)
        k = i % (D // L)
        zero_v[r, pl.ds(k * L, L)] = jnp.zeros((L,), jnp.float32)
        return 0
    lax.fori_loop(0, ZR * (D // L), zbody, 0)
    for kk in range(arows // ZR):
        pltpu.sync_copy(zero_v.at[:, pl.ds(0, DH)],
                        acc0.at[pl.ds(rows0 + kk * ZR, ZR)])
        pltpu.sync_copy(zero_v.at[:, pl.ds(DH, DH)],
                        acc1.at[pl.ds(rows0 + kk * ZR, ZR)])

    banks = [(wrep_a, v0a, v1a, idx_a, sem_la, sem_sa),
             (wrep_b, v0b, v1b, idx_b, sem_lb, sem_sb)]

    def issue_loads(t, bank):
        wrep_v, v0, v1, idx_v, sem_l, _ = bank
        rbase = base + t * CH
        return [
            pltpu.async_copy(
                value_hbm.at[pl.ds(rbase, CH), pl.ds(0, DH)], v0, sem_l),
            pltpu.async_copy(
                value_hbm.at[pl.ds(rbase, CH), pl.ds(DH, DH)], v1, sem_l),
            pltpu.async_copy(wrep_hbm.at[pl.ds(rbase, CH)], wrep_v, sem_l),
            pltpu.async_copy(gidx_hbm.at[pl.ds(rbase, CH)], idx_v, sem_l),
        ]

    plsc.subcore_barrier()

    pend_loads = {0: issue_loads(0, banks[0])}
    pend_scat = {}
    for t in range(NCH):
        bank = banks[t % 2]
        wrep_v, v0, v1, idx_v, _, sem_s = bank
        for cpy in pend_loads.pop(t):
            cpy.wait()
        if t + 1 < NCH:
            nxt = banks[(t + 1) % 2]
            for cpy in pend_scat.pop(t - 1, ()):
                cpy.wait()
            pend_loads[t + 1] = issue_loads(t + 1, nxt)

        def rbody(r, _):
            wrow = wrep_v[r, :]
            for k in range(DH // L):
                v0[r, pl.ds(k * L, L)] = v0[r, pl.ds(k * L, L)] * wrow
                v1[r, pl.ds(k * L, L)] = v1[r, pl.ds(k * L, L)] * wrow
            return 0
        lax.fori_loop(0, CH, rbody, 0)

        pend_scat[t] = [
            pltpu.async_copy(v0, acc0.at[idx_v], sem_s, add=True),
            pltpu.async_copy(v1, acc1.at[idx_v], sem_s, add=True),
        ]

    for t in sorted(pend_scat):
        for cpy in pend_scat[t]:
            cpy.wait()

    plsc.subcore_barrier()
    pltpu.sync_copy(acc0.at[pl.ds(rows0, arows)],
                    out_hbm.at[c, pl.ds(rows0, arows), pl.ds(0, DH)])
    pltpu.sync_copy(acc1.at[pl.ds(rows0, arows)],
                    out_hbm.at[c, pl.ds(rows0, arows), pl.ds(DH, DH)])


_scatter_call = functools.partial(
    pl.kernel,
    out_type=jax.ShapeDtypeStruct((NC, M, D), jnp.float32),
    mesh=plsc.VectorSubcoreMesh(
        core_axis_name="c", subcore_axis_name="s",
        num_cores=NC, num_subcores=NS),
    scratch_types=[
        pltpu.VMEM((CH, L), jnp.float32),
        pltpu.VMEM((CH, L), jnp.float32),
        pltpu.VMEM((CH, DH), jnp.float32),
        pltpu.VMEM((CH, DH), jnp.float32),
        pltpu.VMEM((CH, DH), jnp.float32),
        pltpu.VMEM((CH, DH), jnp.float32),
        pltpu.VMEM((CH,), jnp.int32),
        pltpu.VMEM((CH,), jnp.int32),
        pltpu.VMEM((ZR, D), jnp.float32),
        pltpu.VMEM_SHARED((M, DH), jnp.float32),
        pltpu.VMEM_SHARED((M, DH), jnp.float32),
        pltpu.SemaphoreType.DMA,
        pltpu.SemaphoreType.DMA,
        pltpu.SemaphoreType.DMA,
        pltpu.SemaphoreType.DMA,
    ],
)(_scatter_body)


def _finalize_body(p_ref, keys_ref, out_ref):
    comb = p_ref[0] + p_ref[1] + keys_ref[...]
    nrm = jnp.sqrt(jnp.sum(comb * comb, axis=1, keepdims=True))
    out_ref[...] = comb / jnp.maximum(nrm, 1e-12)


_finalize_call = pl.pallas_call(
    _finalize_body,
    out_shape=jax.ShapeDtypeStruct((M, D), jnp.float32),
)


def kernel(keys, query, value):
    gidx3, wrep3 = _stats_call(keys, query)
    gidx = gidx3.reshape(N)
    wrep = wrep3.reshape(N, L)
    partials = _scatter_call(gidx, wrep, value)
    return _finalize_call(partials, keys)


# speedup vs baseline: 1.1277x; 1.1025x over previous
import functools

import jax
import jax.numpy as jnp
from jax import lax
from jax.experimental import pallas as pl
from jax.experimental.pallas import tpu as pltpu
from jax.experimental.pallas import tpu_sc as plsc

M = 1024
D = 256
N = 16384

NC = 2
NS = 16
L = 16
NW = NC * NS
RPW = N // NW
CH = 128
NCH = RPW // CH

TN = 2048
NT = N // TN


def _stats_body(keys_ref, q_ref, gidx_ref, wrep_ref, rm_s, g_s, cm_s):
    i = pl.program_id(0)

    @pl.when(i < NT)
    def _():
        sc = lax.dot_general(
            keys_ref[...], q_ref[...],
            dimension_numbers=(((1,), (1,)), ((), ())),
            preferred_element_type=jnp.float32,
        )
        rm = jnp.max(sc, axis=0)
        gi = jnp.argmax(sc, axis=0).astype(jnp.int32)
        gidx_ref[...] = gi.reshape(1, 1, TN)
        rm_s[i, :] = rm
        g_s[i, :] = gi
        cm = jnp.max(sc, axis=1).reshape(M, 1)

        @pl.when(i == 0)
        def _():
            cm_s[...] = cm

        @pl.when(i > 0)
        def _():
            cm_s[...] = jnp.maximum(cm_s[...], cm)

    @pl.when(i >= NT)
    def _():
        t = i - NT
        rm = rm_s[t, :]
        g = g_s[t, :]
        rows = lax.broadcasted_iota(jnp.int32, (M, TN), 0)
        cm_bc = jnp.broadcast_to(cm_s[...], (M, TN))
        cmg = jnp.max(jnp.where(rows == g[None, :], cm_bc, -jnp.inf), axis=0)
        w = jnp.exp(rm - cmg)
        wrep_ref[...] = jnp.broadcast_to(w[:, None], (TN, L)).reshape(1, TN, L)


_stats_call = pl.pallas_call(
    _stats_body,
    grid=(2 * NT,),
    in_specs=[
        pl.BlockSpec((M, D), lambda i: (0, 0)),
        pl.BlockSpec((TN, D), lambda i: (jnp.minimum(i, NT - 1), 0)),
    ],
    out_specs=[
        pl.BlockSpec((1, 1, TN),
                     lambda i: (jnp.minimum(i, NT - 1), 0, 0)),
        pl.BlockSpec((1, TN, L),
                     lambda i: (jnp.maximum(i - NT, 0), 0, 0)),
    ],
    out_shape=[
        jax.ShapeDtypeStruct((NT, 1, TN), jnp.int32),
        jax.ShapeDtypeStruct((NT, TN, L), jnp.float32),
    ],
    scratch_shapes=[
        pltpu.VMEM((NT, TN), jnp.float32),
        pltpu.VMEM((NT, TN), jnp.int32),
        pltpu.VMEM((M, 1), jnp.float32),
    ],
)


DH = D // 2
ZR = 16


def _scatter_body(gidx_hbm, wrep_hbm, value_hbm, out_hbm,
                  wrep_a, wrep_b, v0a, v0b, v1a, v1b, idx_a, idx_b, zero_v,
                  acc0, acc1, sem_la, sem_lb, sem_sa, sem_sb):
    c = lax.axis_index("c")
    s = lax.axis_index("s")
    wid = s * NC + c
    base = wid * RPW
    arows = M // NS
    rows0 = s * arows

    def zbody(i, _):
        r = i // (D // L)
        k = i % (D // L)
        zero_v[r, pl.ds(k * L, L)] = jnp.zeros((L,), jnp.float32)
        return 0
    lax.fori_loop(0, ZR * (D // L), zbody, 0)
    for kk in range(arows // ZR):
        pltpu.sync_copy(zero_v.at[:, pl.ds(0, DH)],
                        acc0.at[pl.ds(rows0 + kk * ZR, ZR)])
        pltpu.sync_copy(zero_v.at[:, pl.ds(DH, DH)],
                        acc1.at[pl.ds(rows0 + kk * ZR, ZR)])

    banks = [(wrep_a, v0a, v1a, idx_a, sem_la, sem_sa),
             (wrep_b, v0b, v1b, idx_b, sem_lb, sem_sb)]

    def issue_loads(t, bank):
        wrep_v, v0, v1, idx_v, sem_l, _ = bank
        rbase = base + t * CH
        return [
            pltpu.async_copy(
                value_hbm.at[pl.ds(rbase, CH), pl.ds(0, DH)], v0, sem_l),
            pltpu.async_copy(
                value_hbm.at[pl.ds(rbase, CH), pl.ds(DH, DH)], v1, sem_l),
            pltpu.async_copy(wrep_hbm.at[pl.ds(rbase, CH)], wrep_v, sem_l),
            pltpu.async_copy(gidx_hbm.at[pl.ds(rbase, CH)], idx_v, sem_l),
        ]

    plsc.subcore_barrier()

    pend_loads = {0: issue_loads(0, banks[0])}
    pend_scat = {}
    for t in range(NCH):
        bank = banks[t % 2]
        wrep_v, v0, v1, idx_v, _, sem_s = bank
        for cpy in pend_loads.pop(t):
            cpy.wait()
        if t + 1 < NCH:
            nxt = banks[(t + 1) % 2]
            for cpy in pend_scat.pop(t - 1, ()):
                cpy.wait()
            pend_loads[t + 1] = issue_loads(t + 1, nxt)

        def rbody(r, _):
            wrow = wrep_v[r, :]
            for k in range(DH // L):
                v0[r, pl.ds(k * L, L)] = v0[r, pl.ds(k * L, L)] * wrow
                v1[r, pl.ds(k * L, L)] = v1[r, pl.ds(k * L, L)] * wrow
            return 0
        lax.fori_loop(0, CH, rbody, 0)

        pend_scat[t] = [
            pltpu.async_copy(v0, acc0.at[idx_v], sem_s, add=True),
            pltpu.async_copy(v1, acc1.at[idx_v], sem_s, add=True),
        ]

    for t in sorted(pend_scat):
        for cpy in pend_scat[t]:
            cpy.wait()

    plsc.subcore_barrier()
    pltpu.sync_copy(acc0.at[pl.ds(rows0, arows)],
                    out_hbm.at[c, pl.ds(rows0, arows), pl.ds(0, DH)])
    pltpu.sync_copy(acc1.at[pl.ds(rows0, arows)],
                    out_hbm.at[c, pl.ds(rows0, arows), pl.ds(DH, DH)])


_scatter_call = functools.partial(
    pl.kernel,
    out_type=jax.ShapeDtypeStruct((NC, M, D), jnp.float32),
    mesh=plsc.VectorSubcoreMesh(
        core_axis_name="c", subcore_axis_name="s",
        num_cores=NC, num_subcores=NS),
    scratch_types=[
        pltpu.VMEM((CH, L), jnp.float32),
        pltpu.VMEM((CH, L), jnp.float32),
        pltpu.VMEM((CH, DH), jnp.float32),
        pltpu.VMEM((CH, DH), jnp.float32),
        pltpu.VMEM((CH, DH), jnp.float32),
        pltpu.VMEM((CH, DH), jnp.float32),
        pltpu.VMEM((CH,), jnp.int32),
        pltpu.VMEM((CH,), jnp.int32),
        pltpu.VMEM((ZR, D), jnp.float32),
        pltpu.VMEM_SHARED((M, DH), jnp.float32),
        pltpu.VMEM_SHARED((M, DH), jnp.float32),
        pltpu.SemaphoreType.DMA,
        pltpu.SemaphoreType.DMA,
        pltpu.SemaphoreType.DMA,
        pltpu.SemaphoreType.DMA,
    ],
)(_scatter_body)


def _finalize_body(p_ref, keys_ref, out_ref):
    comb = p_ref[0] + p_ref[1] + keys_ref[...]
    nrm = jnp.sqrt(jnp.sum(comb * comb, axis=1, keepdims=True))
    out_ref[...] = comb / jnp.maximum(nrm, 1e-12)


_finalize_call = pl.pallas_call(
    _finalize_body,
    out_shape=jax.ShapeDtypeStruct((M, D), jnp.float32),
)


def kernel(keys, query, value):
    gidx3, wrep3 = _stats_call(keys, query)
    gidx = gidx3.reshape(N)
    wrep = wrep3.reshape(N, L)
    partials = _scatter_call(gidx, wrep, value)
    return _finalize_call(partials, keys)
